# baseline (speedup 1.0000x reference)
import jax
import jax.numpy as jnp
from jax import lax
from jax.experimental import pallas as pl
from jax.experimental.pallas import tpu as pltpu
from jax.experimental.pallas import tpu_sc as plsc

_E = 768
_H = _E // 2
_M = 16


def _row_lookup_sc(idx_hbm, im_hbm, row_hbm, idx_s):
    @pl.when(lax.axis_index("c") == 0)
    def _():
        pltpu.sync_copy(idx_hbm, idx_s)
        pltpu.sync_copy(im_hbm.at[pl.ds(idx_s[0], 1)], row_hbm)


def _image_row(idx, image2d):
    mesh = plsc.ScalarSubcoreMesh(axis_name="c", num_cores=1)
    return pl.kernel(
        _row_lookup_sc,
        mesh=mesh,
        out_type=jax.ShapeDtypeStruct((1, _H), jnp.float32),
        scratch_types=[
            pltpu.SMEM((1,), jnp.int32),
        ],
    )(idx, image2d)


def _add_body(x_ref, sp_ref, row_ref, o_ref):
    h = sp_ref.shape[-1]
    o_ref[:, :, :h] = x_ref[:, :, :h] + sp_ref[:]
    o_ref[:, :, h:] = x_ref[:, :, h:] + row_ref[0][None, None, :]


def kernel(x, image_idx, spatial_pos_embed, image_pos_embed):
    B, P, E = x.shape
    idx = jnp.asarray(image_idx, jnp.int32).reshape(1)
    row = _image_row(idx, image_pos_embed.reshape(_M, _H))
    bb = 4
    return pl.pallas_call(
        _add_body,
        grid=(B // bb,),
        in_specs=[
            pl.BlockSpec((bb, P, E), lambda b: (b, 0, 0)),
            pl.BlockSpec((1, P, _H), lambda b: (0, 0, 0)),
            pl.BlockSpec((1, _H), lambda b: (0, 0)),
        ],
        out_specs=pl.BlockSpec((bb, P, E), lambda b: (b, 0, 0)),
        out_shape=jax.ShapeDtypeStruct((B, P, E), x.dtype),
        compiler_params=pltpu.CompilerParams(
            dimension_semantics=("arbitrary",),
        ),
    )(x, spatial_pos_embed, row)
